# baseline (speedup 1.0000x reference)
import jax
import jax.numpy as jnp
from jax import lax
from jax.experimental import pallas as pl
from jax.experimental.pallas import tpu as pltpu
from jax.experimental.pallas import tpu_sc as plsc

_NC = 2
_NW = 32
_BATCH = 16384
_BPW = _BATCH // _NW
_D = 64
_NCHUNK = 4
_CH = 128
_PASS = 256


def _afmp_body(da_hbm, db_hbm, emb_hbm, bias_hbm, par_hbm, out_hbm,
               idx_a, idx_b, rows_a, rows_b, bia, bib, par_v, out_v,
               sem, bsem):
    wid = lax.axis_index("s") * _NC + lax.axis_index("c")

    pltpu.sync_copy(da_hbm.at[pl.ds(wid * _NCHUNK, _NCHUNK)], idx_a)
    pltpu.sync_copy(db_hbm.at[pl.ds(wid * _NCHUNK, _NCHUNK)], idx_b)
    pltpu.sync_copy(par_hbm, par_v)

    bias_copies = []
    for j in range(_NCHUNK):
        bias_copies.append(pltpu.async_copy(
            bias_hbm.at[idx_a.at[j]], bia.at[pl.ds(j * _CH, _CH)], bsem))
        bias_copies.append(pltpu.async_copy(
            bias_hbm.at[idx_b.at[j]], bib.at[pl.ds(j * _CH, _CH)], bsem))

    w = [par_v[pl.ds(k * 16, 16)] for k in range(4)]
    wb = par_v[pl.ds(64, 16)]
    b0 = par_v[pl.ds(80, 16)]
    iota16 = lax.iota(jnp.int32, 16)

    for c in bias_copies:
        c.wait()

    for p in range(2):
        def fire_body(t, carry):
            j = p * 2 + t // 8
            v = t % 8
            iva = idx_a[j, pl.ds(v * 16, 16)]
            ivb = idx_b[j, pl.ds(v * 16, 16)]
            base = t * 16
            for l in range(16):
                pltpu.async_copy(emb_hbm.at[pl.ds(iva[l], 1)],
                                 rows_a.at[pl.ds(base + l, 1)], sem)
                pltpu.async_copy(emb_hbm.at[pl.ds(ivb[l], 1)],
                                 rows_b.at[pl.ds(base + l, 1)], sem)
            return carry
        lax.fori_loop(0, _PASS // 16, fire_body, 0)

        pltpu.make_async_copy(emb_hbm.at[pl.ds(0, _PASS)], rows_a, sem).wait()
        pltpu.make_async_copy(emb_hbm.at[pl.ds(0, _PASS)], rows_b, sem).wait()

        def group_body(g, carry):
            acc = jnp.zeros((16,), jnp.float32)
            for r16 in range(16):
                row = g * 16 + r16
                s = rows_a[row, pl.ds(0, 16)] * rows_b[row, pl.ds(0, 16)] * w[0]
                for k in range(1, 4):
                    sl = pl.ds(k * 16, 16)
                    s = s + rows_a[row, sl] * rows_b[row, sl] * w[k]
                t0 = (s[0] + s[1]) + (s[2] + s[3])
                t1 = (s[4] + s[5]) + (s[6] + s[7])
                t2 = (s[8] + s[9]) + (s[10] + s[11])
                t3 = (s[12] + s[13]) + (s[14] + s[15])
                total = (t0 + t1) + (t2 + t3)
                acc = jnp.where(iota16 == r16, total, acc)
            gg = p * 16 + g
            sa = bia[pl.ds(gg * 16, 16)]
            sb = bib[pl.ds(gg * 16, 16)]
            x = acc + b0 + (sa + sb) * wb
            out_v[gg // 8, pl.ds((gg % 8) * 16, 16)] = 1.0 / (1.0 + jnp.exp(-x))
            return carry
        lax.fori_loop(0, _PASS // 16, group_body, 0)

    pltpu.sync_copy(out_v, out_hbm.at[pl.ds(wid * 4, 4)])


_mesh = plsc.VectorSubcoreMesh(core_axis_name="c", subcore_axis_name="s")
_run = pl.kernel(
    _afmp_body,
    out_type=jax.ShapeDtypeStruct((_BATCH // 128, 128), jnp.float32),
    mesh=_mesh,
    scratch_types=[
        pltpu.VMEM((_NCHUNK, _CH), jnp.int32),
        pltpu.VMEM((_NCHUNK, _CH), jnp.int32),
        pltpu.VMEM((_PASS, _D), jnp.float32),
        pltpu.VMEM((_PASS, _D), jnp.float32),
        pltpu.VMEM((_BPW,), jnp.float32),
        pltpu.VMEM((_BPW,), jnp.float32),
        pltpu.VMEM((96,), jnp.float32),
        pltpu.VMEM((4, 128), jnp.float32),
        pltpu.SemaphoreType.DMA,
        pltpu.SemaphoreType.DMA,
    ],
)


def kernel(drug_a, drug_b, emb_table, bias_table, dense_w, dense_b):
    da = drug_a.astype(jnp.int32).reshape(_BATCH // 128, 128)
    db = drug_b.astype(jnp.int32).reshape(_BATCH // 128, 128)
    bias1d = bias_table.reshape(-1)
    params = jnp.concatenate([
        dense_w[:_D, 0],
        jnp.broadcast_to(dense_w[_D, 0], (16,)),
        jnp.broadcast_to(dense_b[0], (16,)),
    ])
    out = _run(da, db, emb_table, bias1d, params)
    return out.reshape(_BATCH, 1)
